# baseline (speedup 1.0000x reference)
import dataclasses
import functools

import jax
import jax.numpy as jnp
from jax import lax
from jax.experimental import pallas as pl
from jax.experimental.pallas import tpu as pltpu
from jax.experimental.pallas import tpu_sc as plsc

B = 16384
D = 64
PACK = 128
NC = 2
NS = 16
NW = NC * NS
B_PER_W = B // NW
CHUNK = 128
N_CHUNKS = B_PER_W // CHUNK
LANES = 16
BU = 1024


def _tp_block(x_ref, o_ref):
    o_ref[:, pl.ds(0, D)] = jnp.transpose(x_ref[...])


def _relayout(table):
    n = table.shape[0]
    t_view = table.T
    return pl.pallas_call(
        _tp_block,
        grid=(pl.cdiv(n, BU),),
        in_specs=[pl.BlockSpec((D, BU), lambda i: (0, i))],
        out_specs=pl.BlockSpec((BU, PACK), lambda i: (i, 0)),
        out_shape=jax.ShapeDtypeStruct((n, PACK), jnp.float32),
        compiler_params=pltpu.CompilerParams(
            dimension_semantics=("parallel",),
        ),
    )(t_view)


def _sc_kernel(uid_hbm, iid_hbm, ut_hbm, it_hbm, out_hbm,
               idx_u, idx_i, urows, irows, outb, sem):
    wid = lax.axis_index("s") * NC + lax.axis_index("c")
    base = wid * B_PER_W
    lane = lax.iota(jnp.int32, LANES)

    for c in range(N_CHUNKS):
        off = base + c * CHUNK
        pltpu.sync_copy(uid_hbm.at[pl.ds(off, CHUNK)], idx_u)
        pltpu.sync_copy(iid_hbm.at[pl.ds(off, CHUNK)], idx_i)
        cp_u = pltpu.async_copy(ut_hbm.at[idx_u], urows, sem)
        cp_i = pltpu.async_copy(it_hbm.at[idx_i], irows, sem)
        cp_u.wait()
        cp_i.wait()

        @pl.loop(0, CHUNK // LANES)
        def _(g):
            r0 = g * LANES
            vec = jnp.zeros((LANES,), jnp.float32)
            for j in range(LANES):
                acc = (urows[r0 + j, pl.ds(0, LANES)]
                       * irows[r0 + j, pl.ds(0, LANES)])
                for k in range(1, D // LANES):
                    acc += (urows[r0 + j, pl.ds(k * LANES, LANES)]
                            * irows[r0 + j, pl.ds(k * LANES, LANES)])
                vec = jnp.where(lane == j, jnp.sum(acc), vec)
            outb[pl.ds(r0, LANES)] = vec

        pltpu.sync_copy(outb, out_hbm.at[pl.ds(off, CHUNK)])


@jax.jit
def kernel(user_ids, item_ids, user_table, item_table):
    ut_p = _relayout(user_table)
    it_p = _relayout(item_table)
    mesh = plsc.VectorSubcoreMesh(core_axis_name="c", subcore_axis_name="s")
    cp = pltpu.CompilerParams()
    if "needs_layout_passes" in pltpu.CompilerParams.__dataclass_fields__:
        cp = dataclasses.replace(cp, needs_layout_passes=False)
    run = pl.kernel(
        _sc_kernel,
        out_type=jax.ShapeDtypeStruct((B,), jnp.float32),
        mesh=mesh,
        scratch_types=[
            pltpu.VMEM((CHUNK,), jnp.int32),
            pltpu.VMEM((CHUNK,), jnp.int32),
            pltpu.VMEM((CHUNK, PACK), jnp.float32),
            pltpu.VMEM((CHUNK, PACK), jnp.float32),
            pltpu.VMEM((CHUNK,), jnp.float32),
            pltpu.SemaphoreType.DMA,
        ],
        compiler_params=cp,
    )
    return run(user_ids.astype(jnp.int32), item_ids.astype(jnp.int32),
               ut_p, it_p)


# speedup vs baseline: 2.1054x; 2.1054x over previous
import dataclasses
import functools

import jax
import jax.numpy as jnp
from jax import lax
from jax.experimental import pallas as pl
from jax.experimental.pallas import tpu as pltpu
from jax.experimental.pallas import tpu_sc as plsc

B = 16384
D = 64
PACK = 128
NC = 2
NS = 16
NW = NC * NS
B_PER_W = B // NW
CHUNK = 128
N_CHUNKS = B_PER_W // CHUNK
LANES = 16
BU = 8192


def _tp_block(x_ref, o_ref):
    t = jnp.transpose(x_ref[...])
    o_ref[...] = jnp.concatenate([t, t], axis=1)


def _relayout(table):
    n = table.shape[0]
    t_view = table.T
    return pl.pallas_call(
        _tp_block,
        grid=(pl.cdiv(n, BU),),
        in_specs=[pl.BlockSpec((D, BU), lambda i: (0, i))],
        out_specs=pl.BlockSpec((BU, PACK), lambda i: (i, 0)),
        out_shape=jax.ShapeDtypeStruct((n, PACK), jnp.float32),
        compiler_params=pltpu.CompilerParams(
            dimension_semantics=("arbitrary",),
        ),
    )(t_view)


def _sc_kernel(uid_hbm, iid_hbm, ut_hbm, it_hbm, out_hbm,
               idx_u, idx_i, urows, irows, outb, sem):
    wid = lax.axis_index("s") * NC + lax.axis_index("c")
    base = wid * B_PER_W
    lane = lax.iota(jnp.int32, LANES)

    for c in range(N_CHUNKS):
        off = base + c * CHUNK
        pltpu.sync_copy(uid_hbm.at[pl.ds(off, CHUNK)], idx_u)
        pltpu.sync_copy(iid_hbm.at[pl.ds(off, CHUNK)], idx_i)
        cp_u = pltpu.async_copy(ut_hbm.at[idx_u], urows, sem)
        cp_i = pltpu.async_copy(it_hbm.at[idx_i], irows, sem)
        cp_u.wait()
        cp_i.wait()

        @pl.loop(0, CHUNK // LANES)
        def _(g):
            r0 = g * LANES
            vec = jnp.zeros((LANES,), jnp.float32)
            for j in range(LANES):
                acc = (urows[r0 + j, pl.ds(0, LANES)]
                       * irows[r0 + j, pl.ds(0, LANES)])
                for k in range(1, D // LANES):
                    acc += (urows[r0 + j, pl.ds(k * LANES, LANES)]
                            * irows[r0 + j, pl.ds(k * LANES, LANES)])
                vec = jnp.where(lane == j, jnp.sum(acc), vec)
            outb[pl.ds(r0, LANES)] = vec

        pltpu.sync_copy(outb, out_hbm.at[pl.ds(off, CHUNK)])


@jax.jit
def kernel(user_ids, item_ids, user_table, item_table):
    ut_p = _relayout(user_table)
    it_p = _relayout(item_table)
    mesh = plsc.VectorSubcoreMesh(core_axis_name="c", subcore_axis_name="s")
    cp = pltpu.CompilerParams()
    if "needs_layout_passes" in pltpu.CompilerParams.__dataclass_fields__:
        cp = dataclasses.replace(cp, needs_layout_passes=False)
    run = pl.kernel(
        _sc_kernel,
        out_type=jax.ShapeDtypeStruct((B,), jnp.float32),
        mesh=mesh,
        scratch_types=[
            pltpu.VMEM((CHUNK,), jnp.int32),
            pltpu.VMEM((CHUNK,), jnp.int32),
            pltpu.VMEM((CHUNK, PACK), jnp.float32),
            pltpu.VMEM((CHUNK, PACK), jnp.float32),
            pltpu.VMEM((CHUNK,), jnp.float32),
            pltpu.SemaphoreType.DMA,
        ],
        compiler_params=cp,
    )
    return run(user_ids.astype(jnp.int32), item_ids.astype(jnp.int32),
               ut_p, it_p)


# speedup vs baseline: 2.3081x; 1.0962x over previous
import dataclasses
import functools

import jax
import jax.numpy as jnp
from jax import lax
from jax.experimental import pallas as pl
from jax.experimental.pallas import tpu as pltpu
from jax.experimental.pallas import tpu_sc as plsc

B = 16384
D = 64
PACK = 128
NC = 2
NS = 16
NW = NC * NS
B_PER_W = B // NW
CHUNK = 128
N_CHUNKS = B_PER_W // CHUNK
LANES = 16
BU = 16384


def _tp_block(x_ref, o_ref):
    t = jnp.transpose(x_ref[...])
    o_ref[...] = jnp.concatenate([t, t], axis=1)


def _relayout(table):
    n = table.shape[0]
    t_view = table.T
    return pl.pallas_call(
        _tp_block,
        grid=(pl.cdiv(n, BU),),
        in_specs=[pl.BlockSpec((D, BU), lambda i: (0, i))],
        out_specs=pl.BlockSpec((BU, PACK), lambda i: (i, 0)),
        out_shape=jax.ShapeDtypeStruct((n, PACK), jnp.float32),
        compiler_params=pltpu.CompilerParams(
            dimension_semantics=("arbitrary",),
        ),
    )(t_view)


def _sc_kernel(uid_hbm, iid_hbm, ut_hbm, it_hbm, out_hbm,
               idx_u, idx_i, urows, irows, outb, sem):
    wid = lax.axis_index("s") * NC + lax.axis_index("c")
    base = wid * B_PER_W
    lane = lax.iota(jnp.int32, LANES)

    for c in range(N_CHUNKS):
        off = base + c * CHUNK
        pltpu.sync_copy(uid_hbm.at[pl.ds(off, CHUNK)], idx_u)
        pltpu.sync_copy(iid_hbm.at[pl.ds(off, CHUNK)], idx_i)
        cp_u = pltpu.async_copy(ut_hbm.at[idx_u], urows, sem)
        cp_i = pltpu.async_copy(it_hbm.at[idx_i], irows, sem)
        cp_u.wait()
        cp_i.wait()

        @pl.loop(0, CHUNK // LANES)
        def _(g):
            r0 = g * LANES
            vec = jnp.zeros((LANES,), jnp.float32)
            for j in range(LANES):
                acc = (urows[r0 + j, pl.ds(0, LANES)]
                       * irows[r0 + j, pl.ds(0, LANES)])
                for k in range(1, D // LANES):
                    acc += (urows[r0 + j, pl.ds(k * LANES, LANES)]
                            * irows[r0 + j, pl.ds(k * LANES, LANES)])
                vec = jnp.where(lane == j, jnp.sum(acc), vec)
            outb[pl.ds(r0, LANES)] = vec

        pltpu.sync_copy(outb, out_hbm.at[pl.ds(off, CHUNK)])


@jax.jit
def kernel(user_ids, item_ids, user_table, item_table):
    ut_p = _relayout(user_table)
    it_p = _relayout(item_table)
    mesh = plsc.VectorSubcoreMesh(core_axis_name="c", subcore_axis_name="s")
    cp = pltpu.CompilerParams()
    if "needs_layout_passes" in pltpu.CompilerParams.__dataclass_fields__:
        cp = dataclasses.replace(cp, needs_layout_passes=False)
    run = pl.kernel(
        _sc_kernel,
        out_type=jax.ShapeDtypeStruct((B,), jnp.float32),
        mesh=mesh,
        scratch_types=[
            pltpu.VMEM((CHUNK,), jnp.int32),
            pltpu.VMEM((CHUNK,), jnp.int32),
            pltpu.VMEM((CHUNK, PACK), jnp.float32),
            pltpu.VMEM((CHUNK, PACK), jnp.float32),
            pltpu.VMEM((CHUNK,), jnp.float32),
            pltpu.SemaphoreType.DMA,
        ],
        compiler_params=cp,
    )
    return run(user_ids.astype(jnp.int32), item_ids.astype(jnp.int32),
               ut_p, it_p)


# speedup vs baseline: 2.3592x; 1.0222x over previous
import dataclasses
import functools

import jax
import jax.numpy as jnp
from jax import lax
from jax.experimental import pallas as pl
from jax.experimental.pallas import tpu as pltpu
from jax.experimental.pallas import tpu_sc as plsc

B = 16384
D = 64
PACK = 128
NC = 2
NS = 16
NW = NC * NS
B_PER_W = B // NW
CHUNK = 128
N_CHUNKS = B_PER_W // CHUNK
LANES = 16
BU = 24576


def _tp_block(x_ref, o_ref):
    t = jnp.transpose(x_ref[...])
    o_ref[...] = jnp.concatenate([t, t], axis=1)


def _relayout(table):
    n = table.shape[0]
    t_view = table.T
    return pl.pallas_call(
        _tp_block,
        grid=(pl.cdiv(n, BU),),
        in_specs=[pl.BlockSpec((D, BU), lambda i: (0, i))],
        out_specs=pl.BlockSpec((BU, PACK), lambda i: (i, 0)),
        out_shape=jax.ShapeDtypeStruct((n, PACK), jnp.float32),
        compiler_params=pltpu.CompilerParams(
            dimension_semantics=("arbitrary",),
        ),
    )(t_view)


def _sc_kernel(uid_hbm, iid_hbm, ut_hbm, it_hbm, out_hbm,
               idx_u, idx_i, urows, irows, outb, sem):
    wid = lax.axis_index("s") * NC + lax.axis_index("c")
    base = wid * B_PER_W
    lane = lax.iota(jnp.int32, LANES)

    for c in range(N_CHUNKS):
        off = base + c * CHUNK
        pltpu.sync_copy(uid_hbm.at[pl.ds(off, CHUNK)], idx_u)
        pltpu.sync_copy(iid_hbm.at[pl.ds(off, CHUNK)], idx_i)
        cp_u = pltpu.async_copy(ut_hbm.at[idx_u], urows, sem)
        cp_i = pltpu.async_copy(it_hbm.at[idx_i], irows, sem)
        cp_u.wait()
        cp_i.wait()

        @pl.loop(0, CHUNK // LANES)
        def _(g):
            r0 = g * LANES
            vec = jnp.zeros((LANES,), jnp.float32)
            for j in range(LANES):
                acc = (urows[r0 + j, pl.ds(0, LANES)]
                       * irows[r0 + j, pl.ds(0, LANES)])
                for k in range(1, D // LANES):
                    acc += (urows[r0 + j, pl.ds(k * LANES, LANES)]
                            * irows[r0 + j, pl.ds(k * LANES, LANES)])
                vec = jnp.where(lane == j, jnp.sum(acc), vec)
            outb[pl.ds(r0, LANES)] = vec

        pltpu.sync_copy(outb, out_hbm.at[pl.ds(off, CHUNK)])


@jax.jit
def kernel(user_ids, item_ids, user_table, item_table):
    ut_p = _relayout(user_table)
    it_p = _relayout(item_table)
    mesh = plsc.VectorSubcoreMesh(core_axis_name="c", subcore_axis_name="s")
    cp = pltpu.CompilerParams()
    if "needs_layout_passes" in pltpu.CompilerParams.__dataclass_fields__:
        cp = dataclasses.replace(cp, needs_layout_passes=False)
    run = pl.kernel(
        _sc_kernel,
        out_type=jax.ShapeDtypeStruct((B,), jnp.float32),
        mesh=mesh,
        scratch_types=[
            pltpu.VMEM((CHUNK,), jnp.int32),
            pltpu.VMEM((CHUNK,), jnp.int32),
            pltpu.VMEM((CHUNK, PACK), jnp.float32),
            pltpu.VMEM((CHUNK, PACK), jnp.float32),
            pltpu.VMEM((CHUNK,), jnp.float32),
            pltpu.SemaphoreType.DMA,
        ],
        compiler_params=cp,
    )
    return run(user_ids.astype(jnp.int32), item_ids.astype(jnp.int32),
               ut_p, it_p)
